# baseline (speedup 1.0000x reference)
import functools

import jax
import jax.numpy as jnp
from jax import lax
from jax.experimental import pallas as pl
from jax.experimental.pallas import tpu as pltpu
from jax.experimental.pallas import tpu_sc as plsc

NSONGS = 100000
DIM = 64
BATCH = 1024
HIST = 200

NC = 2
NS = 16
NW = NC * NS

NCHUNK = 4
CB = BATCH // NCHUNK

CHUNKS = ((0, 128), (128, 72))

_sc_mesh = plsc.VectorSubcoreMesh(core_axis_name="c", subcore_axis_name="s")


def _make_pool(rows):
    rows_per_w = rows // NW
    idx_per_w = rows_per_w * HIST

    @functools.partial(
        pl.kernel,
        out_type=jax.ShapeDtypeStruct((rows, DIM), jnp.float32),
        mesh=_sc_mesh,
        scratch_types=[
            pltpu.VMEM((idx_per_w,), jnp.int32),
            pltpu.VMEM((2, HIST, DIM), jnp.float32),
            pltpu.VMEM((rows_per_w, DIM), jnp.float32),
            pltpu.SemaphoreType.DMA,
            pltpu.SemaphoreType.DMA,
        ],
        compiler_params=pltpu.CompilerParams(use_tc_tiling_on_sc=False),
    )
    def pool(x_hbm, table_hbm, h_hbm, idx_v, rows_v, hbuf_v, sem0, sem1):
        wid = lax.axis_index("s") * NC + lax.axis_index("c")
        base = wid * rows_per_w
        zero = jnp.zeros((16,), jnp.float32)
        sems = (sem0, sem1)
        ione = jnp.ones((16,), jnp.int32)

        pltpu.sync_copy(x_hbm.at[pl.ds(wid * idx_per_w, idx_per_w)], idx_v)

        def inc_body(g, _):
            for u in range(4):
                off = g * 64 + u * 16
                idx_v[pl.ds(off, 16)] = idx_v[pl.ds(off, 16)] + ione
            return 0

        lax.fori_loop(0, idx_per_w // 64, inc_body, 0)

        def fetch_issue(local_row, buf):
            for off, n in CHUNKS:
                pltpu.async_copy(
                    table_hbm.at[idx_v.at[pl.ds(local_row * HIST + off, n)]],
                    rows_v.at[buf].at[pl.ds(off, n)],
                    sems[buf],
                )

        def drain(local_row, buf):
            for off, n in CHUNKS:
                pltpu.make_async_copy(
                    table_hbm.at[idx_v.at[pl.ds(local_row * HIST + off, n)]],
                    rows_v.at[buf].at[pl.ds(off, n)],
                    sems[buf],
                ).wait()

        def reduce_into(local_row, buf):
            rb = rows_v.at[buf]

            def t_body(j, accs):
                a = list(accs)
                t0 = j * 8
                for u in range(8):
                    for c in range(4):
                        a[c] = a[c] + rb[t0 + u, pl.ds(c * 16, 16)]
                return tuple(a)

            accs = lax.fori_loop(0, HIST // 8, t_body, (zero,) * 4)
            for c in range(4):
                hbuf_v[local_row, pl.ds(c * 16, 16)] = accs[c]

        fetch_issue(0, 0)

        def g_body(g, _):
            fetch_issue(2 * g + 1, 1)
            drain(2 * g, 0)
            reduce_into(2 * g, 0)

            @pl.when(g < rows_per_w // 2 - 1)
            def _():
                fetch_issue(2 * g + 2, 0)

            drain(2 * g + 1, 1)
            reduce_into(2 * g + 1, 1)
            return 0

        lax.fori_loop(0, rows_per_w // 2, g_body, 0)
        pltpu.sync_copy(hbuf_v, h_hbm.at[pl.ds(base, rows_per_w)])

    return pool


_pool_chunk = _make_pool(CB)

BN = 4096


def _decode_body(h_ref, w_ref, b_ref, o_ref):
    o_ref[...] = (
        lax.dot_general(
            h_ref[...],
            w_ref[...],
            (((1,), (1,)), ((), ())),
            preferred_element_type=jnp.float32,
        )
        + b_ref[...]
    )


def _decode_body_alias(o_alias_ref, h_ref, w_ref, b_ref, o_ref):
    del o_alias_ref
    _decode_body(h_ref, w_ref, b_ref, o_ref)


def _decode_chunk(out_buf, h_c, inv_w, inv_b2, c):
    nblk = pl.cdiv(NSONGS, BN)
    hwb_specs = [
        pl.BlockSpec((CB, DIM), lambda j: (0, 0)),
        pl.BlockSpec((BN, DIM), lambda j: (j, 0)),
        pl.BlockSpec((1, BN), lambda j: (0, j)),
    ]
    if out_buf is None:
        return pl.pallas_call(
            _decode_body,
            grid=(nblk,),
            in_specs=hwb_specs,
            out_specs=pl.BlockSpec((CB, BN), lambda j, c=c: (c, j)),
            out_shape=jax.ShapeDtypeStruct((BATCH, NSONGS), jnp.float32),
        )(h_c, inv_w, inv_b2)
    return pl.pallas_call(
        _decode_body_alias,
        grid=(nblk,),
        in_specs=[pl.BlockSpec(memory_space=pl.ANY)] + hwb_specs,
        out_specs=pl.BlockSpec((CB, BN), lambda j, c=c: (c, j)),
        out_shape=jax.ShapeDtypeStruct((BATCH, NSONGS), jnp.float32),
        input_output_aliases={0: 0},
    )(out_buf, h_c, inv_w, inv_b2)


def kernel(x, emb_weight, inv_w, inv_b):
    xi = x.astype(jnp.int32)
    inv_b2 = inv_b.reshape(1, NSONGS)
    hs = [
        _pool_chunk(
            lax.slice(xi, (c * CB, 0), ((c + 1) * CB, HIST)).reshape(CB * HIST),
            emb_weight,
        )
        for c in range(NCHUNK)
    ]
    out = None
    for c in range(NCHUNK):
        out = _decode_chunk(out, hs[c], inv_w, inv_b2, c)
    return out
